# baseline (speedup 1.0000x reference)
import jax
import jax.numpy as jnp
from jax import lax
from jax.experimental import pallas as pl
from jax.experimental.pallas import tpu as pltpu, tpu_sc as plsc

_NT = 32
_SC_PARAMS = pltpu.CompilerParams(needs_layout_passes=False)


def _mesh():
    return plsc.VectorSubcoreMesh(core_axis_name="c", subcore_axis_name="s")


def _wid():
    return lax.axis_index("s") * 2 + lax.axis_index("c")


_Z16 = lambda: jnp.zeros((16,), jnp.float32)


def kernel(x, edge_index, edge_attr, W1, b1, W2, b2):
    src_e = edge_index[0]
    dst_e = edge_index[1]
    N, D_in = x.shape
    E = edge_attr.shape[0]
    D_h = W1.shape[1]
    D_out = W2.shape[1]
    f32 = jnp.float32

    EPT = E // _NT

    def deg_body(dst_hbm, w_hbm, out_hbm, dst_v, w_v, acc_v):
        wid = _wid()
        base = wid * EPT

        @plsc.parallel_loop(0, N, 16, unroll=8)
        def _zero(i):
            acc_v[pl.ds(i, 16)] = _Z16()

        pltpu.sync_copy(dst_hbm.at[pl.ds(base, EPT)], dst_v)
        pltpu.sync_copy(w_hbm.at[pl.ds(base, EPT)], w_v)

        @plsc.parallel_loop(0, EPT, 16, unroll=8)
        def _scat(j):
            d = dst_v[pl.ds(j, 16)]
            wv = w_v[pl.ds(j, 16)]
            plsc.addupdate_scatter(acc_v, [d], wv)

        pltpu.sync_copy(acc_v, out_hbm.at[pl.ds(wid * N, N)])

    deg_part = pl.kernel(
        deg_body,
        out_type=jax.ShapeDtypeStruct((_NT * N,), f32),
        mesh=_mesh(),
        scratch_types=[
            pltpu.VMEM((EPT,), jnp.int32),
            pltpu.VMEM((EPT,), f32),
            pltpu.VMEM((N,), f32),
        ],
        compiler_params=_SC_PARAMS,
    )(dst_e, edge_attr)

    def tc1(deg_ref, x_ref, w1_ref, dinv_ref, h1t_ref):
        deg = jnp.sum(deg_ref[...], axis=0) + 1.0
        dinv_ref[...] = lax.rsqrt(deg)
        h1t_ref[...] = lax.dot_general(
            w1_ref[...], x_ref[...], (((0,), (1,)), ((), ())),
            preferred_element_type=f32)

    dinv, h1T = pl.pallas_call(
        tc1,
        out_shape=(
            jax.ShapeDtypeStruct((N,), f32),
            jax.ShapeDtypeStruct((D_h, N), f32),
        ),
    )(deg_part.reshape(_NT, N), x, W1)

    def norm_body(src_hbm, dst_hbm, w_hbm, dinv_hbm, nrm_hbm,
                  src_v, dst_v, w_v, dinv_v, nrm_v):
        wid = _wid()
        base = wid * EPT
        pltpu.sync_copy(dinv_hbm, dinv_v)
        pltpu.sync_copy(src_hbm.at[pl.ds(base, EPT)], src_v)
        pltpu.sync_copy(dst_hbm.at[pl.ds(base, EPT)], dst_v)
        pltpu.sync_copy(w_hbm.at[pl.ds(base, EPT)], w_v)

        @plsc.parallel_loop(0, EPT, 16, unroll=8)
        def _nrm(j):
            sl = pl.ds(j, 16)
            s = src_v[sl]
            d = dst_v[sl]
            wv = w_v[sl]
            nrm_v[sl] = plsc.load_gather(dinv_v, [s]) * wv * plsc.load_gather(dinv_v, [d])

        pltpu.sync_copy(nrm_v, nrm_hbm.at[pl.ds(base, EPT)])

    norm = pl.kernel(
        norm_body,
        out_type=jax.ShapeDtypeStruct((E,), f32),
        mesh=_mesh(),
        scratch_types=[
            pltpu.VMEM((EPT,), jnp.int32),
            pltpu.VMEM((EPT,), jnp.int32),
            pltpu.VMEM((EPT,), f32),
            pltpu.VMEM((N,), f32),
            pltpu.VMEM((EPT,), f32),
        ],
        compiler_params=_SC_PARAMS,
    )(src_e, dst_e, edge_attr, dinv)

    def make_agg(D, F, n_groups, n_reps, chunk):
        eps = E // n_reps
        nch = eps // chunk
        g_mask = n_groups - 1
        r_shift = n_groups.bit_length() - 1
        FN = F * N

        def body(src_hbm, dst_hbm, nrm_hbm, ht_hbm, out_hbm,
                 h_v, acc_v, src_v, dst_v, nrm_v):
            wid = _wid()
            g = wid & g_mask
            r = wid >> r_shift
            pltpu.sync_copy(ht_hbm.at[pl.ds(g * FN, FN)], h_v)

            @plsc.parallel_loop(0, FN, 16, unroll=8)
            def _zero(i):
                acc_v[pl.ds(i, 16)] = _Z16()

            ebase = r * eps

            def chunk_step(ch, _):
                off = ebase + ch * chunk
                pltpu.sync_copy(src_hbm.at[pl.ds(off, chunk)], src_v)
                pltpu.sync_copy(dst_hbm.at[pl.ds(off, chunk)], dst_v)
                pltpu.sync_copy(nrm_hbm.at[pl.ds(off, chunk)], nrm_v)

                @plsc.parallel_loop(0, chunk, 16, unroll=8)
                def _agg(j):
                    sl = pl.ds(j, 16)
                    s = src_v[sl]
                    d = dst_v[sl]
                    nm = nrm_v[sl]
                    for f in range(F):
                        v = plsc.load_gather(h_v.at[pl.ds(f * N, N)], [s])
                        plsc.addupdate_scatter(acc_v.at[pl.ds(f * N, N)], [d], nm * v)
                return _

            lax.fori_loop(0, nch, chunk_step, None)
            pltpu.sync_copy(acc_v, out_hbm.at[pl.ds(r * (D * N) + g * FN, FN)])

        return pl.kernel(
            body,
            out_type=jax.ShapeDtypeStruct((n_reps * D * N,), f32),
            mesh=_mesh(),
            scratch_types=[
                pltpu.VMEM((FN,), f32),
                pltpu.VMEM((FN,), f32),
                pltpu.VMEM((chunk,), jnp.int32),
                pltpu.VMEM((chunk,), jnp.int32),
                pltpu.VMEM((chunk,), f32),
            ],
            compiler_params=_SC_PARAMS,
        )

    p1 = make_agg(D_h, 4, 16, 2, 10000)(src_e, dst_e, norm, h1T.reshape(-1))

    def tc2(p_ref, h1t_ref, dinv_ref, w2_ref, b1_ref, h2t_ref):
        dinv = dinv_ref[...]
        o = p_ref[0] + p_ref[1] + (dinv * dinv) * h1t_ref[...]
        o = o + b1_ref[...][:, None]
        z = jnp.maximum(o, 0.0)
        h2t_ref[...] = lax.dot_general(
            w2_ref[...], z, (((0,), (0,)), ((), ())),
            preferred_element_type=f32)

    h2T = pl.pallas_call(
        tc2,
        out_shape=jax.ShapeDtypeStruct((D_out, N), f32),
    )(p1.reshape(2, D_h, N), h1T, dinv, W2, b1)

    p2 = make_agg(D_out, 5, 8, 4, 4000)(src_e, dst_e, norm, h2T.reshape(-1))

    def tc3(p_ref, h2t_ref, dinv_ref, b2_ref, out_ref):
        dinv = dinv_ref[...]
        o = p_ref[0] + p_ref[1] + p_ref[2] + p_ref[3]
        o = o + (dinv * dinv) * h2t_ref[...] + b2_ref[...][:, None]
        m = jnp.max(o, axis=0, keepdims=True)
        e = jnp.exp(o - m)
        sm = e / jnp.sum(e, axis=0, keepdims=True)
        out_ref[...] = jnp.transpose(sm, (1, 0))

    out = pl.pallas_call(
        tc3,
        out_shape=jax.ShapeDtypeStruct((N, D_out), f32),
    )(p2.reshape(4, D_out, N), h2T, dinv, b2)
    return out


# speedup vs baseline: 1.3643x; 1.3643x over previous
import jax
import jax.numpy as jnp
from jax import lax
from jax.experimental import pallas as pl
from jax.experimental.pallas import tpu as pltpu, tpu_sc as plsc

_NT = 32
_SC_PARAMS = pltpu.CompilerParams(needs_layout_passes=False)


def _mesh():
    return plsc.VectorSubcoreMesh(core_axis_name="c", subcore_axis_name="s")


def _wid():
    return lax.axis_index("s") * 2 + lax.axis_index("c")


_Z16 = lambda: jnp.zeros((16,), jnp.float32)


def kernel(x, edge_index, edge_attr, W1, b1, W2, b2):
    src_e = edge_index[0]
    dst_e = edge_index[1]
    N, D_in = x.shape
    E = edge_attr.shape[0]
    D_h = W1.shape[1]
    D_out = W2.shape[1]
    f32 = jnp.float32

    EPT = E // _NT

    def deg_body(dst_hbm, w_hbm, out_hbm, dst_v, w_v, acc_v):
        wid = _wid()
        base = wid * EPT

        @plsc.parallel_loop(0, N, 16, unroll=8)
        def _zero(i):
            acc_v[pl.ds(i, 16)] = _Z16()

        pltpu.sync_copy(dst_hbm.at[pl.ds(base, EPT)], dst_v)
        pltpu.sync_copy(w_hbm.at[pl.ds(base, EPT)], w_v)

        @plsc.parallel_loop(0, EPT, 16, unroll=8)
        def _scat(j):
            d = dst_v[pl.ds(j, 16)]
            wv = w_v[pl.ds(j, 16)]
            plsc.addupdate_scatter(acc_v, [d], wv)

        pltpu.sync_copy(acc_v, out_hbm.at[pl.ds(wid * N, N)])

    deg_part = pl.kernel(
        deg_body,
        out_type=jax.ShapeDtypeStruct((_NT * N,), f32),
        mesh=_mesh(),
        scratch_types=[
            pltpu.VMEM((EPT,), jnp.int32),
            pltpu.VMEM((EPT,), f32),
            pltpu.VMEM((N,), f32),
        ],
        compiler_params=_SC_PARAMS,
    )(dst_e, edge_attr)

    def tc1(deg_ref, x_ref, w1_ref, dinv_ref, h1t_ref):
        deg = jnp.sum(deg_ref[...], axis=0) + 1.0
        dinv_ref[...] = lax.rsqrt(deg)
        h1t_ref[...] = lax.dot_general(
            w1_ref[...], x_ref[...], (((0,), (1,)), ((), ())),
            preferred_element_type=f32)

    dinv, h1T = pl.pallas_call(
        tc1,
        out_shape=(
            jax.ShapeDtypeStruct((N,), f32),
            jax.ShapeDtypeStruct((D_h, N), f32),
        ),
    )(deg_part.reshape(_NT, N), x, W1)

    def norm_body(src_hbm, dst_hbm, w_hbm, dinv_hbm, nrm_hbm,
                  src_v, dst_v, w_v, dinv_v, nrm_v):
        wid = _wid()
        base = wid * EPT
        pltpu.sync_copy(dinv_hbm, dinv_v)
        pltpu.sync_copy(src_hbm.at[pl.ds(base, EPT)], src_v)
        pltpu.sync_copy(dst_hbm.at[pl.ds(base, EPT)], dst_v)
        pltpu.sync_copy(w_hbm.at[pl.ds(base, EPT)], w_v)

        @plsc.parallel_loop(0, EPT, 16, unroll=8)
        def _nrm(j):
            sl = pl.ds(j, 16)
            s = src_v[sl]
            d = dst_v[sl]
            wv = w_v[sl]
            nrm_v[sl] = plsc.load_gather(dinv_v, [s]) * wv * plsc.load_gather(dinv_v, [d])

        pltpu.sync_copy(nrm_v, nrm_hbm.at[pl.ds(base, EPT)])

    norm = pl.kernel(
        norm_body,
        out_type=jax.ShapeDtypeStruct((E,), f32),
        mesh=_mesh(),
        scratch_types=[
            pltpu.VMEM((EPT,), jnp.int32),
            pltpu.VMEM((EPT,), jnp.int32),
            pltpu.VMEM((EPT,), f32),
            pltpu.VMEM((N,), f32),
            pltpu.VMEM((EPT,), f32),
        ],
        compiler_params=_SC_PARAMS,
    )(src_e, dst_e, edge_attr, dinv)

    def make_agg(D, F, n_groups, n_reps, chunk):
        eps = E // n_reps
        nch = eps // chunk
        g_mask = n_groups - 1
        r_shift = n_groups.bit_length() - 1
        FN = F * N

        npairs = nch // 2
        assert nch % 2 == 0

        def body(src_hbm, dst_hbm, nrm_hbm, ht_hbm, out_hbm,
                 h_v, acc_v,
                 src_v0, dst_v0, nrm_v0, src_v1, dst_v1, nrm_v1,
                 ss0, sd0, sn0, ss1, sd1, sn1):
            wid = _wid()
            g = wid & g_mask
            r = wid >> r_shift
            ebase = r * eps
            E_tot = src_hbm.shape[0]

            def issue(bufs, sems, ch):
                off = jnp.minimum(ebase + ch * chunk, E_tot - chunk)
                pltpu.async_copy(src_hbm.at[pl.ds(off, chunk)], bufs[0], sems[0])
                pltpu.async_copy(dst_hbm.at[pl.ds(off, chunk)], bufs[1], sems[1])
                pltpu.async_copy(nrm_hbm.at[pl.ds(off, chunk)], bufs[2], sems[2])

            def wait(bufs, sems):
                pltpu.make_async_copy(src_hbm.at[pl.ds(0, chunk)], bufs[0], sems[0]).wait()
                pltpu.make_async_copy(dst_hbm.at[pl.ds(0, chunk)], bufs[1], sems[1]).wait()
                pltpu.make_async_copy(nrm_hbm.at[pl.ds(0, chunk)], bufs[2], sems[2]).wait()

            def process(bufs):
                src_v, dst_v, nrm_v = bufs

                @plsc.parallel_loop(0, chunk, 16, unroll=4)
                def _agg(j):
                    sl = pl.ds(j, 16)
                    s = src_v[sl]
                    d = dst_v[sl]
                    nm = nrm_v[sl]
                    for f in range(F):
                        v = plsc.load_gather(h_v.at[pl.ds(f * N, N)], [s])
                        plsc.addupdate_scatter(acc_v.at[pl.ds(f * N, N)], [d], nm * v)

            b0 = (src_v0, dst_v0, nrm_v0)
            b1 = (src_v1, dst_v1, nrm_v1)
            s0 = (ss0, sd0, sn0)
            s1 = (ss1, sd1, sn1)

            issue(b0, s0, 0)
            pltpu.sync_copy(ht_hbm.at[pl.ds(g * FN, FN)], h_v)

            @plsc.parallel_loop(0, FN, 16, unroll=8)
            def _zero(i):
                acc_v[pl.ds(i, 16)] = _Z16()

            def pair_step(cp, _):
                ch0 = cp * 2
                issue(b1, s1, ch0 + 1)
                wait(b0, s0)
                process(b0)
                issue(b0, s0, ch0 + 2)
                wait(b1, s1)
                process(b1)
                return _

            lax.fori_loop(0, npairs, pair_step, None)
            wait(b0, s0)
            pltpu.sync_copy(acc_v, out_hbm.at[pl.ds(r * (D * N) + g * FN, FN)])

        return pl.kernel(
            body,
            out_type=jax.ShapeDtypeStruct((n_reps * D * N,), f32),
            mesh=_mesh(),
            scratch_types=[
                pltpu.VMEM((FN,), f32),
                pltpu.VMEM((FN,), f32),
                pltpu.VMEM((chunk,), jnp.int32),
                pltpu.VMEM((chunk,), jnp.int32),
                pltpu.VMEM((chunk,), f32),
                pltpu.VMEM((chunk,), jnp.int32),
                pltpu.VMEM((chunk,), jnp.int32),
                pltpu.VMEM((chunk,), f32),
                pltpu.SemaphoreType.DMA,
                pltpu.SemaphoreType.DMA,
                pltpu.SemaphoreType.DMA,
                pltpu.SemaphoreType.DMA,
                pltpu.SemaphoreType.DMA,
                pltpu.SemaphoreType.DMA,
            ],
            compiler_params=_SC_PARAMS,
        )

    p1 = make_agg(D_h, 4, 16, 2, 8000)(src_e, dst_e, norm, h1T.reshape(-1))

    def tc2(p_ref, h1t_ref, dinv_ref, w2_ref, b1_ref, h2t_ref):
        dinv = dinv_ref[...]
        o = p_ref[0] + p_ref[1] + (dinv * dinv) * h1t_ref[...]
        o = o + b1_ref[...][:, None]
        z = jnp.maximum(o, 0.0)
        h2t_ref[...] = lax.dot_general(
            w2_ref[...], z, (((0,), (0,)), ((), ())),
            preferred_element_type=f32)

    h2T = pl.pallas_call(
        tc2,
        out_shape=jax.ShapeDtypeStruct((D_out, N), f32),
    )(p1.reshape(2, D_h, N), h1T, dinv, W2, b1)

    p2 = make_agg(D_out, 5, 8, 4, 4000)(src_e, dst_e, norm, h2T.reshape(-1))

    def tc3(p_ref, h2t_ref, dinv_ref, b2_ref, out_ref):
        dinv = dinv_ref[...]
        o = p_ref[0] + p_ref[1] + p_ref[2] + p_ref[3]
        o = o + (dinv * dinv) * h2t_ref[...] + b2_ref[...][:, None]
        m = jnp.max(o, axis=0, keepdims=True)
        e = jnp.exp(o - m)
        sm = e / jnp.sum(e, axis=0, keepdims=True)
        out_ref[...] = jnp.transpose(sm, (1, 0))

    out = pl.pallas_call(
        tc3,
        out_shape=jax.ShapeDtypeStruct((N, D_out), f32),
    )(p2.reshape(4, D_out, N), h2T, dinv, b2)
    return out


# speedup vs baseline: 1.3704x; 1.0045x over previous
import jax
import jax.numpy as jnp
from jax import lax
from jax.experimental import pallas as pl
from jax.experimental.pallas import tpu as pltpu, tpu_sc as plsc

_NT = 32
_SC_PARAMS = pltpu.CompilerParams(needs_layout_passes=False)


def _mesh():
    return plsc.VectorSubcoreMesh(core_axis_name="c", subcore_axis_name="s")


def _wid():
    return lax.axis_index("s") * 2 + lax.axis_index("c")


_Z16 = lambda: jnp.zeros((16,), jnp.float32)


def kernel(x, edge_index, edge_attr, W1, b1, W2, b2):
    src_e = edge_index[0]
    dst_e = edge_index[1]
    N, D_in = x.shape
    E = edge_attr.shape[0]
    D_h = W1.shape[1]
    D_out = W2.shape[1]
    f32 = jnp.float32

    EPT = E // _NT

    def deg_body(dst_hbm, w_hbm, out_hbm, dst_v, w_v, acc_v):
        wid = _wid()
        base = wid * EPT

        @plsc.parallel_loop(0, N, 16, unroll=8)
        def _zero(i):
            acc_v[pl.ds(i, 16)] = _Z16()

        pltpu.sync_copy(dst_hbm.at[pl.ds(base, EPT)], dst_v)
        pltpu.sync_copy(w_hbm.at[pl.ds(base, EPT)], w_v)

        @plsc.parallel_loop(0, EPT, 16, unroll=8)
        def _scat(j):
            d = dst_v[pl.ds(j, 16)]
            wv = w_v[pl.ds(j, 16)]
            plsc.addupdate_scatter(acc_v, [d], wv)

        pltpu.sync_copy(acc_v, out_hbm.at[pl.ds(wid * N, N)])

    deg_part = pl.kernel(
        deg_body,
        out_type=jax.ShapeDtypeStruct((_NT * N,), f32),
        mesh=_mesh(),
        scratch_types=[
            pltpu.VMEM((EPT,), jnp.int32),
            pltpu.VMEM((EPT,), f32),
            pltpu.VMEM((N,), f32),
        ],
        compiler_params=_SC_PARAMS,
    )(dst_e, edge_attr)

    def tc1a(x_ref, w1_ref, h1t_ref):
        h1t_ref[...] = lax.dot_general(
            w1_ref[...], x_ref[...], (((0,), (1,)), ((), ())),
            preferred_element_type=f32)

    h1T = pl.pallas_call(
        tc1a,
        out_shape=jax.ShapeDtypeStruct((D_h, N), f32),
    )(x, W1)

    def tc1b(deg_ref, dinv_ref):
        deg = jnp.sum(deg_ref[...], axis=0) + 1.0
        dinv_ref[...] = lax.rsqrt(deg)

    dinv = pl.pallas_call(
        tc1b,
        out_shape=jax.ShapeDtypeStruct((N,), f32),
    )(deg_part.reshape(_NT, N))

    def norm_body(src_hbm, dst_hbm, w_hbm, dinv_hbm, nrm_hbm,
                  src_v, dst_v, w_v, dinv_v, nrm_v):
        wid = _wid()
        base = wid * EPT
        pltpu.sync_copy(dinv_hbm, dinv_v)
        pltpu.sync_copy(src_hbm.at[pl.ds(base, EPT)], src_v)
        pltpu.sync_copy(dst_hbm.at[pl.ds(base, EPT)], dst_v)
        pltpu.sync_copy(w_hbm.at[pl.ds(base, EPT)], w_v)

        @plsc.parallel_loop(0, EPT, 16, unroll=8)
        def _nrm(j):
            sl = pl.ds(j, 16)
            s = src_v[sl]
            d = dst_v[sl]
            wv = w_v[sl]
            nrm_v[sl] = plsc.load_gather(dinv_v, [s]) * wv * plsc.load_gather(dinv_v, [d])

        pltpu.sync_copy(nrm_v, nrm_hbm.at[pl.ds(base, EPT)])

    norm = pl.kernel(
        norm_body,
        out_type=jax.ShapeDtypeStruct((E,), f32),
        mesh=_mesh(),
        scratch_types=[
            pltpu.VMEM((EPT,), jnp.int32),
            pltpu.VMEM((EPT,), jnp.int32),
            pltpu.VMEM((EPT,), f32),
            pltpu.VMEM((N,), f32),
            pltpu.VMEM((EPT,), f32),
        ],
        compiler_params=_SC_PARAMS,
    )(src_e, dst_e, edge_attr, dinv)

    def make_agg(D, F, n_groups, n_reps, chunk):
        eps = E // n_reps
        nch = eps // chunk
        g_mask = n_groups - 1
        r_shift = n_groups.bit_length() - 1
        FN = F * N

        npairs = nch // 2
        assert nch % 2 == 0

        def body(src_hbm, dst_hbm, nrm_hbm, ht_hbm, out_hbm,
                 h_v, acc_v,
                 src_v0, dst_v0, nrm_v0, src_v1, dst_v1, nrm_v1,
                 ss0, sd0, sn0, ss1, sd1, sn1):
            wid = _wid()
            g = wid & g_mask
            r = wid >> r_shift
            ebase = r * eps
            E_tot = src_hbm.shape[0]

            def issue(bufs, sems, ch):
                off = jnp.minimum(ebase + ch * chunk, E_tot - chunk)
                pltpu.async_copy(src_hbm.at[pl.ds(off, chunk)], bufs[0], sems[0])
                pltpu.async_copy(dst_hbm.at[pl.ds(off, chunk)], bufs[1], sems[1])
                pltpu.async_copy(nrm_hbm.at[pl.ds(off, chunk)], bufs[2], sems[2])

            def wait(bufs, sems):
                pltpu.make_async_copy(src_hbm.at[pl.ds(0, chunk)], bufs[0], sems[0]).wait()
                pltpu.make_async_copy(dst_hbm.at[pl.ds(0, chunk)], bufs[1], sems[1]).wait()
                pltpu.make_async_copy(nrm_hbm.at[pl.ds(0, chunk)], bufs[2], sems[2]).wait()

            def process(bufs):
                src_v, dst_v, nrm_v = bufs

                @plsc.parallel_loop(0, chunk, 16, unroll=4)
                def _agg(j):
                    sl = pl.ds(j, 16)
                    s = src_v[sl]
                    d = dst_v[sl]
                    nm = nrm_v[sl]
                    for f in range(F):
                        v = plsc.load_gather(h_v.at[pl.ds(f * N, N)], [s])
                        plsc.addupdate_scatter(acc_v.at[pl.ds(f * N, N)], [d], nm * v)

            b0 = (src_v0, dst_v0, nrm_v0)
            b1 = (src_v1, dst_v1, nrm_v1)
            s0 = (ss0, sd0, sn0)
            s1 = (ss1, sd1, sn1)

            issue(b0, s0, 0)
            pltpu.sync_copy(ht_hbm.at[pl.ds(g * FN, FN)], h_v)

            @plsc.parallel_loop(0, FN, 16, unroll=8)
            def _zero(i):
                acc_v[pl.ds(i, 16)] = _Z16()

            def pair_step(cp, _):
                ch0 = cp * 2
                issue(b1, s1, ch0 + 1)
                wait(b0, s0)
                process(b0)
                issue(b0, s0, ch0 + 2)
                wait(b1, s1)
                process(b1)
                return _

            lax.fori_loop(0, npairs, pair_step, None)
            wait(b0, s0)
            pltpu.sync_copy(acc_v, out_hbm.at[pl.ds(r * (D * N) + g * FN, FN)])

        return pl.kernel(
            body,
            out_type=jax.ShapeDtypeStruct((n_reps * D * N,), f32),
            mesh=_mesh(),
            scratch_types=[
                pltpu.VMEM((FN,), f32),
                pltpu.VMEM((FN,), f32),
                pltpu.VMEM((chunk,), jnp.int32),
                pltpu.VMEM((chunk,), jnp.int32),
                pltpu.VMEM((chunk,), f32),
                pltpu.VMEM((chunk,), jnp.int32),
                pltpu.VMEM((chunk,), jnp.int32),
                pltpu.VMEM((chunk,), f32),
                pltpu.SemaphoreType.DMA,
                pltpu.SemaphoreType.DMA,
                pltpu.SemaphoreType.DMA,
                pltpu.SemaphoreType.DMA,
                pltpu.SemaphoreType.DMA,
                pltpu.SemaphoreType.DMA,
            ],
            compiler_params=_SC_PARAMS,
        )

    p1 = make_agg(D_h, 4, 16, 2, 8000)(src_e, dst_e, norm, h1T.reshape(-1))

    def tc2(p_ref, h1t_ref, dinv_ref, w2_ref, b1_ref, h2t_ref):
        dinv = dinv_ref[...]
        o = p_ref[0] + p_ref[1] + (dinv * dinv) * h1t_ref[...]
        o = o + b1_ref[...][:, None]
        z = jnp.maximum(o, 0.0)
        h2t_ref[...] = lax.dot_general(
            w2_ref[...], z, (((0,), (0,)), ((), ())),
            preferred_element_type=f32)

    h2T = pl.pallas_call(
        tc2,
        out_shape=jax.ShapeDtypeStruct((D_out, N), f32),
    )(p1.reshape(2, D_h, N), h1T, dinv, W2, b1)

    p2 = make_agg(D_out, 5, 8, 4, 4000)(src_e, dst_e, norm, h2T.reshape(-1))

    def tc3(p_ref, h2t_ref, dinv_ref, b2_ref, out_ref):
        dinv = dinv_ref[...]
        o = p_ref[0] + p_ref[1] + p_ref[2] + p_ref[3]
        o = o + (dinv * dinv) * h2t_ref[...] + b2_ref[...][:, None]
        m = jnp.max(o, axis=0, keepdims=True)
        e = jnp.exp(o - m)
        sm = e / jnp.sum(e, axis=0, keepdims=True)
        out_ref[...] = jnp.transpose(sm, (1, 0))

    out = pl.pallas_call(
        tc3,
        out_shape=jax.ShapeDtypeStruct((N, D_out), f32),
    )(p2.reshape(4, D_out, N), h2T, dinv, b2)
    return out


# speedup vs baseline: 1.4156x; 1.0330x over previous
import jax
import jax.numpy as jnp
from jax import lax
from jax.experimental import pallas as pl
from jax.experimental.pallas import tpu as pltpu, tpu_sc as plsc

_NT = 32
_SC_PARAMS = pltpu.CompilerParams(needs_layout_passes=False)


def _mesh():
    return plsc.VectorSubcoreMesh(core_axis_name="c", subcore_axis_name="s")


def _wid():
    return lax.axis_index("s") * 2 + lax.axis_index("c")


_Z16 = lambda: jnp.zeros((16,), jnp.float32)


def kernel(x, edge_index, edge_attr, W1, b1, W2, b2):
    src_e = edge_index[0]
    dst_e = edge_index[1]
    N, D_in = x.shape
    E = edge_attr.shape[0]
    D_h = W1.shape[1]
    D_out = W2.shape[1]
    f32 = jnp.float32

    EPT = E // _NT

    def deg_body(dst_hbm, w_hbm, out_hbm, dst_v, w_v, acc_v):
        wid = _wid()
        base = wid * EPT

        @plsc.parallel_loop(0, N, 16, unroll=8)
        def _zero(i):
            acc_v[pl.ds(i, 16)] = _Z16()

        pltpu.sync_copy(dst_hbm.at[pl.ds(base, EPT)], dst_v)
        pltpu.sync_copy(w_hbm.at[pl.ds(base, EPT)], w_v)

        @plsc.parallel_loop(0, EPT, 16, unroll=8)
        def _scat(j):
            d = dst_v[pl.ds(j, 16)]
            wv = w_v[pl.ds(j, 16)]
            plsc.addupdate_scatter(acc_v, [d], wv)

        pltpu.sync_copy(acc_v, out_hbm.at[pl.ds(wid * N, N)])

    deg_part = pl.kernel(
        deg_body,
        out_type=jax.ShapeDtypeStruct((_NT * N,), f32),
        mesh=_mesh(),
        scratch_types=[
            pltpu.VMEM((EPT,), jnp.int32),
            pltpu.VMEM((EPT,), f32),
            pltpu.VMEM((N,), f32),
        ],
        compiler_params=_SC_PARAMS,
    )(dst_e, edge_attr)

    def tc1a(x_ref, w1_ref, h1t_ref):
        h1t_ref[...] = lax.dot_general(
            w1_ref[...], x_ref[...], (((0,), (1,)), ((), ())),
            preferred_element_type=f32)

    h1T = pl.pallas_call(
        tc1a,
        out_shape=jax.ShapeDtypeStruct((D_h, N), f32),
    )(x, W1)

    def tc1b(deg_ref, dinv_ref):
        deg = jnp.sum(deg_ref[...], axis=0) + 1.0
        dinv_ref[...] = lax.rsqrt(deg)

    dinv = pl.pallas_call(
        tc1b,
        out_shape=jax.ShapeDtypeStruct((N,), f32),
    )(deg_part.reshape(_NT, N))

    def norm_body(src_hbm, dst_hbm, w_hbm, dinv_hbm, nrm_hbm, pk_hbm,
                  src_v, dst_v, w_v, dinv_v, nrm_v, pk_v):
        wid = _wid()
        base = wid * EPT
        pltpu.sync_copy(dinv_hbm, dinv_v)
        pltpu.sync_copy(src_hbm.at[pl.ds(base, EPT)], src_v)
        pltpu.sync_copy(dst_hbm.at[pl.ds(base, EPT)], dst_v)
        pltpu.sync_copy(w_hbm.at[pl.ds(base, EPT)], w_v)

        @plsc.parallel_loop(0, EPT, 16, unroll=8)
        def _nrm(j):
            sl = pl.ds(j, 16)
            s = src_v[sl]
            d = dst_v[sl]
            wv = w_v[sl]
            nrm_v[sl] = plsc.load_gather(dinv_v, [s]) * wv * plsc.load_gather(dinv_v, [d])
            pk_v[sl] = (s << 16) | d

        pltpu.sync_copy(nrm_v, nrm_hbm.at[pl.ds(base, EPT)])
        pltpu.sync_copy(pk_v, pk_hbm.at[pl.ds(base, EPT)])

    norm, packed = pl.kernel(
        norm_body,
        out_type=(jax.ShapeDtypeStruct((E,), f32),
                  jax.ShapeDtypeStruct((E,), jnp.int32)),
        mesh=_mesh(),
        scratch_types=[
            pltpu.VMEM((EPT,), jnp.int32),
            pltpu.VMEM((EPT,), jnp.int32),
            pltpu.VMEM((EPT,), f32),
            pltpu.VMEM((N,), f32),
            pltpu.VMEM((EPT,), f32),
            pltpu.VMEM((EPT,), jnp.int32),
        ],
        compiler_params=_SC_PARAMS,
    )(src_e, dst_e, edge_attr, dinv)

    def make_agg(D, F, n_groups, n_reps, chunk):
        eps = E // n_reps
        nch = eps // chunk
        g_mask = n_groups - 1
        r_shift = n_groups.bit_length() - 1
        FN = F * N

        npairs = nch // 2
        assert nch % 2 == 0

        def body(pk_hbm, nrm_hbm, ht_hbm, out_hbm,
                 h_v, acc_v,
                 pk_v0, nrm_v0, pk_v1, nrm_v1,
                 sp0, sn0, sp1, sn1):
            wid = _wid()
            g = wid & g_mask
            r = wid >> r_shift
            ebase = r * eps
            E_tot = pk_hbm.shape[0]

            def issue(bufs, sems, ch):
                off = jnp.minimum(ebase + ch * chunk, E_tot - chunk)
                pltpu.async_copy(pk_hbm.at[pl.ds(off, chunk)], bufs[0], sems[0])
                pltpu.async_copy(nrm_hbm.at[pl.ds(off, chunk)], bufs[1], sems[1])

            def wait(bufs, sems):
                pltpu.make_async_copy(pk_hbm.at[pl.ds(0, chunk)], bufs[0], sems[0]).wait()
                pltpu.make_async_copy(nrm_hbm.at[pl.ds(0, chunk)], bufs[1], sems[1]).wait()

            def process(bufs):
                pk_v, nrm_v = bufs

                @plsc.parallel_loop(0, chunk, 16, unroll=4)
                def _agg(j):
                    sl = pl.ds(j, 16)
                    p = pk_v[sl]
                    s = p >> 16
                    d = p & 0xFFFF
                    nm = nrm_v[sl]
                    for f in range(F):
                        v = plsc.load_gather(h_v.at[pl.ds(f * N, N)], [s])
                        plsc.addupdate_scatter(acc_v.at[pl.ds(f * N, N)], [d], nm * v)

            b0 = (pk_v0, nrm_v0)
            b1 = (pk_v1, nrm_v1)
            s0 = (sp0, sn0)
            s1 = (sp1, sn1)

            issue(b0, s0, 0)
            pltpu.sync_copy(ht_hbm.at[pl.ds(g * FN, FN)], h_v)

            @plsc.parallel_loop(0, FN, 16, unroll=8)
            def _zero(i):
                acc_v[pl.ds(i, 16)] = _Z16()

            def pair_step(cp, _):
                ch0 = cp * 2
                issue(b1, s1, ch0 + 1)
                wait(b0, s0)
                process(b0)
                issue(b0, s0, ch0 + 2)
                wait(b1, s1)
                process(b1)
                return _

            lax.fori_loop(0, npairs, pair_step, None)
            wait(b0, s0)
            pltpu.sync_copy(acc_v, out_hbm.at[pl.ds(r * (D * N) + g * FN, FN)])

        return pl.kernel(
            body,
            out_type=jax.ShapeDtypeStruct((n_reps * D * N,), f32),
            mesh=_mesh(),
            scratch_types=[
                pltpu.VMEM((FN,), f32),
                pltpu.VMEM((FN,), f32),
                pltpu.VMEM((chunk,), jnp.int32),
                pltpu.VMEM((chunk,), f32),
                pltpu.VMEM((chunk,), jnp.int32),
                pltpu.VMEM((chunk,), f32),
                pltpu.SemaphoreType.DMA,
                pltpu.SemaphoreType.DMA,
                pltpu.SemaphoreType.DMA,
                pltpu.SemaphoreType.DMA,
            ],
            compiler_params=_SC_PARAMS,
        )

    p1 = make_agg(D_h, 4, 16, 2, 8000)(packed, norm, h1T.reshape(-1))

    def tc2(p_ref, h1t_ref, dinv_ref, w2_ref, b1_ref, h2t_ref):
        dinv = dinv_ref[...]
        o = p_ref[0] + p_ref[1] + (dinv * dinv) * h1t_ref[...]
        o = o + b1_ref[...][:, None]
        z = jnp.maximum(o, 0.0)
        h2t_ref[...] = lax.dot_general(
            w2_ref[...], z, (((0,), (0,)), ((), ())),
            preferred_element_type=f32)

    h2T = pl.pallas_call(
        tc2,
        out_shape=jax.ShapeDtypeStruct((D_out, N), f32),
    )(p1.reshape(2, D_h, N), h1T, dinv, W2, b1)

    p2 = make_agg(D_out, 5, 8, 4, 4000)(packed, norm, h2T.reshape(-1))

    def tc3(p_ref, h2t_ref, dinv_ref, b2_ref, out_ref):
        dinv = dinv_ref[...]
        o = p_ref[0] + p_ref[1] + p_ref[2] + p_ref[3]
        o = o + (dinv * dinv) * h2t_ref[...] + b2_ref[...][:, None]
        m = jnp.max(o, axis=0, keepdims=True)
        e = jnp.exp(o - m)
        sm = e / jnp.sum(e, axis=0, keepdims=True)
        out_ref[...] = jnp.transpose(sm, (1, 0))

    out = pl.pallas_call(
        tc3,
        out_shape=jax.ShapeDtypeStruct((N, D_out), f32),
    )(p2.reshape(4, D_out, N), h2T, dinv, b2)
    return out
